# baseline (speedup 1.0000x reference)
import functools

import jax
import jax.numpy as jnp
from jax import lax
from jax.experimental import pallas as pl
from jax.experimental.pallas import tpu as pltpu
from jax.experimental.pallas import tpu_sc as plsc

F32 = jnp.float32

N = 10000
E = 320000
D_IN = 128
H = 64
HW = 128
DEPTH = 4

NC = 2
NS = 16
NW = NC * NS

CHUNK = 128
KG = 2
KS = 2
EPT = -(-E // (NW * CHUNK * KG * KS)) * (CHUNK * KG * KS)
E_PAD = EPT * NW
NCH = EPT // CHUNK
NSS_G = NCH // KG
NSS_S = NCH // KS
NSH = ((N + 1) + NS * 8 - 1) // (NS * 8) * (NS * 8)
ROWS_PER_TILE = NSH // NS

_PREC = lax.Precision.HIGHEST


def _silu(v):
    return v * jax.nn.sigmoid(v)


def _dot(a, b):
    return jnp.dot(a, b, preferred_element_type=F32, precision=_PREC)


def _widen(v):
    return jnp.concatenate([v, jnp.zeros_like(v)], axis=1)


@functools.cache
def _sc_mesh():
    return plsc.VectorSubcoreMesh(
        core_axis_name="c", subcore_axis_name="s", num_cores=NC, num_subcores=NS)


@functools.cache
def _sc_gather_kernel():
    @functools.partial(
        pl.kernel,
        out_type=(
            jax.ShapeDtypeStruct((E_PAD, HW), F32),
            jax.ShapeDtypeStruct((E_PAD, HW), F32),
        ),
        mesh=_sc_mesh(),
    scratch_types=[
            pltpu.VMEM((CHUNK,), jnp.int32),
            pltpu.VMEM((CHUNK, HW), F32),
            pltpu.SemaphoreType.DMA,
        ],
    )
    def gather(feat, src3, dst3, hsrc, hdst, idx_v, rows_v, sem):
        wid = lax.axis_index("c") * NS + lax.axis_index("s")
        base0 = wid * EPT

        def body(i, carry):
            base = base0 + i * CHUNK
            pltpu.sync_copy(src3.at[wid, i], idx_v)
            pltpu.async_copy(feat.at[idx_v], rows_v, sem).wait()
            pltpu.sync_copy(rows_v, hsrc.at[pl.ds(base, CHUNK)])
            pltpu.sync_copy(dst3.at[wid, i], idx_v)
            pltpu.async_copy(feat.at[idx_v], rows_v, sem).wait()
            pltpu.sync_copy(rows_v, hdst.at[pl.ds(base, CHUNK)])
            return carry

        lax.fori_loop(0, NCH, body, 0)

    return gather


def _sc_gather(feat, src3, dst3):
    return _sc_gather_kernel()(feat, src3, dst3)


@functools.cache
def _sc_scatter_kernel():
    @functools.partial(
        pl.kernel,
        out_type=jax.ShapeDtypeStruct((NC, NSH, HW), F32),
        mesh=_sc_mesh(),
        scratch_types=[
            pltpu.VMEM((NCH, CHUNK), jnp.int32),
            pltpu.VMEM((KS, CHUNK, HW), F32),
            pltpu.VMEM_SHARED((NSH, HW), F32),
        ] + [pltpu.SemaphoreType.DMA] * KS,
    )
    def scatter(m, dst3, zeros, out, idx_all, rows, acc, *sl):
        cid = lax.axis_index("c")
        sid = lax.axis_index("s")
        r0 = sid * ROWS_PER_TILE
        pltpu.sync_copy(zeros.at[pl.ds(r0, ROWS_PER_TILE)],
                        acc.at[pl.ds(r0, ROWS_PER_TILE)])
        plsc.subcore_barrier()

        wid = cid * NS + sid
        base0 = wid * EPT
        pltpu.sync_copy(dst3.at[wid], idx_all)

        def l_start(b, c):
            pltpu.async_copy(m.at[pl.ds(base0 + c * CHUNK, CHUNK)], rows.at[b], sl[b])

        def l_wait(b):
            pltpu.make_async_copy(m.at[pl.ds(base0, CHUNK)], rows.at[b], sl[b]).wait()

        for b in range(KS):
            l_start(b, b)

        def body(s, carry):
            c0 = s * KS
            for b in range(KS):
                l_wait(b)
                pltpu.sync_copy(rows.at[b], acc.at[idx_all.at[c0 + b]], add=True)
                l_start(b, jnp.minimum(c0 + KS + b, NCH - 1))
            return carry

        lax.fori_loop(0, NSS_S, body, 0)
        for b in range(KS):
            l_wait(b)
        plsc.subcore_barrier()
        pltpu.sync_copy(acc.at[pl.ds(r0, ROWS_PER_TILE)],
                        out.at[cid, pl.ds(r0, ROWS_PER_TILE)])

    return scatter


def _sc_scatter(m, dst3, zeros):
    return _sc_scatter_kernel()(m, dst3, zeros)


def _in_mlp_body(x_ref, w_ref, b_ref, o_ref):
    o_ref[...] = _widen(_silu(_dot(x_ref[...], w_ref[...]) + b_ref[...]))


def _tc_in_mlp(x, w, b):
    return pl.pallas_call(
        _in_mlp_body,
        out_shape=jax.ShapeDtypeStruct((N, HW), F32),
    )(x, w, b)


BE = 2048


def _edge_mlp_body(hs_ref, hd_ref, d_ref, w1a, w1b, w1c, b1, w2, b2, wa, ba, o_ref):
    z = (_dot(hs_ref[...], w1a[...]) + _dot(hd_ref[...], w1b[...])
         + d_ref[...] * w1c[...] + b1[...])
    z = _silu(z)
    mm = _silu(_dot(z, w2[...]) + b2[...])
    w = jax.nn.sigmoid(_dot(mm, wa[...]) + ba[...])
    o_ref[...] = _widen(mm * w)


def _tc_edge_mlp(hs, hd, dpad, w1a, w1b, w1c, b1, w2, b2, wa, ba):
    grid = (E_PAD // BE,)
    full = lambda a: pl.BlockSpec(a.shape, lambda i: (0,) * a.ndim)
    return pl.pallas_call(
        _edge_mlp_body,
        grid=grid,
        in_specs=[
            pl.BlockSpec((BE, HW), lambda i: (i, 0)),
            pl.BlockSpec((BE, HW), lambda i: (i, 0)),
            pl.BlockSpec((BE, 1), lambda i: (i, 0)),
            full(w1a), full(w1b), full(w1c), full(b1),
            full(w2), full(b2), full(wa), full(ba),
        ],
        out_specs=pl.BlockSpec((BE, HW), lambda i: (i, 0)),
        out_shape=jax.ShapeDtypeStruct((E_PAD, HW), F32),
        compiler_params=pltpu.CompilerParams(dimension_semantics=("arbitrary",)),
    )(hs, hd, dpad, w1a, w1b, w1c, b1, w2, b2, wa, ba)


def _update_body(p_ref, feat_ref, w1, b1, w2, b2, o_ref):
    agg = p_ref[0, :N, :H] + p_ref[1, :N, :H]
    u = _silu(_dot(agg, w1[...]) + b1[...])
    o_ref[...] = _widen(_dot(u, w2[...]) + b2[...] + feat_ref[:, :H])


def _tc_update(partial, feat, w1, b1, w2, b2):
    return pl.pallas_call(
        _update_body,
        out_shape=jax.ShapeDtypeStruct((N, HW), F32),
    )(partial, feat, w1, b1, w2, b2)


def _final_body(feat_ref, nw1, nb1, nw2, nb2, ow1, ob1, ow2, ob2, o_ref):
    f = _silu(_dot(feat_ref[:, :H], nw1[...]) + nb1[...])
    f = _dot(f, nw2[...]) + nb2[...]
    s = jnp.sum(f, axis=0, keepdims=True)
    mx = jnp.max(f, axis=0, keepdims=True)
    r = jnp.concatenate([s, s * (1.0 / N), mx], axis=1)
    h = jax.nn.relu(_dot(r, ow1[...]) + ob1[...])
    o_ref[...] = _dot(h, ow2[...]) + ob2[...]


def _tc_final(feat, nw1, nb1, nw2, nb2, ow1, ob1, ow2, ob2):
    t = ow2.shape[1]
    return pl.pallas_call(
        _final_body,
        out_shape=jax.ShapeDtypeStruct((1, t), F32),
    )(feat, nw1, nb1, nw2, nb2, ow1, ob1, ow2, ob2)


def kernel(x, edge_index, d, W_in, b_in, msg_W1, msg_b1, msg_W2, msg_b2,
           att_W, att_b, upd_W1, upd_b1, upd_W2, upd_b2, now_W1, now_b1,
           now_W2, now_b2, out_W1, out_b1, out_W2, out_b2):
    ei = edge_index.astype(jnp.int32)
    src3 = jnp.pad(ei[0], (0, E_PAD - E)).reshape(NW, NCH, CHUNK)
    dst3 = jnp.pad(ei[1], (0, E_PAD - E), constant_values=N).reshape(NW, NCH, CHUNK)
    dpad = jnp.pad(d, ((0, E_PAD - E), (0, 0)))
    zeros = jnp.zeros((NSH, HW), F32)

    feat = _tc_in_mlp(x, W_in, b_in[None, :])

    for l in range(DEPTH):
        hsrc, hdst = _sc_gather(feat, src3, dst3)
        w1a = jnp.pad(msg_W1[l, :H], ((0, HW - H), (0, 0)))
        w1b = jnp.pad(msg_W1[l, H:2 * H], ((0, HW - H), (0, 0)))
        m = _tc_edge_mlp(
            hsrc, hdst, dpad,
            w1a, w1b, msg_W1[l, 2 * H:2 * H + 1],
            msg_b1[l][None, :], msg_W2[l], msg_b2[l][None, :],
            att_W[l], att_b[l][None, :],
        )
        partial = _sc_scatter(m, dst3, zeros)
        feat = _tc_update(partial, feat, upd_W1[l], upd_b1[l][None, :],
                          upd_W2[l], upd_b2[l][None, :])

    return _tc_final(feat, now_W1, now_b1[None, :], now_W2, now_b2[None, :],
                     out_W1, out_b1[None, :], out_W2, out_b2[None, :])


# speedup vs baseline: 1.5263x; 1.5263x over previous
import functools

import jax
import jax.numpy as jnp
from jax import lax
from jax.experimental import pallas as pl
from jax.experimental.pallas import tpu as pltpu
from jax.experimental.pallas import tpu_sc as plsc

F32 = jnp.float32

N = 10000
E = 320000
D_IN = 128
H = 64
HW = 128
DEPTH = 4

NC = 2
NS = 16
NW = NC * NS

CHUNK = 128
EPT = -(-E // (NW * CHUNK * 2)) * (CHUNK * 2)
E_PAD = EPT * NW
NCH = EPT // CHUNK
NSH = ((N + 1) + NS * 8 - 1) // (NS * 8) * (NS * 8)
ROWS_PER_TILE = NSH // NS

_PREC = lax.Precision.DEFAULT


def _silu(v):
    return v * jax.nn.sigmoid(v)


def _dot(a, b):
    return jnp.dot(a, b, preferred_element_type=F32, precision=_PREC)


@functools.cache
def _sc_mesh():
    return plsc.VectorSubcoreMesh(
        core_axis_name="c", subcore_axis_name="s", num_cores=NC, num_subcores=NS)


@functools.cache
def _sc_gather_kernel():
    @functools.partial(
        pl.kernel,
        out_type=jax.ShapeDtypeStruct((E_PAD, HW), F32),
        mesh=_sc_mesh(),
        scratch_types=[
            pltpu.VMEM((CHUNK,), jnp.int32),
            pltpu.VMEM((2, CHUNK, HW), F32),
            pltpu.VMEM((2, CHUNK, HW), F32),
            pltpu.SemaphoreType.DMA,
            pltpu.SemaphoreType.DMA,
            pltpu.SemaphoreType.DMA,
        ],
    )
    def gather(tab, src, dst, z, idx_v, rows, zbuf, sem, semw0, semw1):
        semw = (semw0, semw1)
        wid = lax.axis_index("c") * NS + lax.axis_index("s")
        base0 = wid * EPT

        def fetch(i):
            base = base0 + i * CHUNK
            pltpu.sync_copy(src.at[pl.ds(base, CHUNK)], idx_v)
            pltpu.async_copy(tab.at[idx_v], rows.at[0], sem).wait()
            pltpu.sync_copy(dst.at[pl.ds(base, CHUNK)], idx_v)
            pltpu.async_copy(tab.at[idx_v], rows.at[1], sem).wait()

        def add_rows(p):
            def row(r, carry):
                for g in range(H // 16):
                    zbuf[p, r, pl.ds(g * 16, 16)] = (
                        rows[0, r, pl.ds(g * 16, 16)]
                        + rows[1, r, pl.ds(H + g * 16, 16)])
                return carry
            lax.fori_loop(0, CHUNK, row, 0)

        def wb_start(p, i):
            pltpu.async_copy(
                zbuf.at[p], z.at[pl.ds(base0 + i * CHUNK, CHUNK)], semw[p])

        def wb_wait(p):
            pltpu.make_async_copy(
                zbuf.at[p], z.at[pl.ds(base0, CHUNK)], semw[p]).wait()

        def step(p, i):
            fetch(i)
            add_rows(p)
            wb_start(p, i)

        step(0, 0)
        step(1, 1)

        def body(k, carry):
            c0 = 2 * k
            wb_wait(0)
            step(0, c0)
            wb_wait(1)
            step(1, c0 + 1)
            return carry

        lax.fori_loop(1, NCH // 2, body, 0)
        wb_wait(0)
        wb_wait(1)

    return gather


def _sc_gather(tab, src, dst):
    return _sc_gather_kernel()(tab, src, dst)


@functools.cache
def _sc_scatter_kernel():
    @functools.partial(
        pl.kernel,
        out_type=jax.ShapeDtypeStruct((NC, NSH, HW), F32),
        mesh=_sc_mesh(),
        scratch_types=[
            pltpu.VMEM((CHUNK,), jnp.int32),
            pltpu.VMEM((CHUNK, HW), F32),
            pltpu.VMEM_SHARED((NSH, HW), F32),
        ],
    )
    def scatter(m, dsti, zeros, out, idx_v, rows_v, acc):
        cid = lax.axis_index("c")
        sid = lax.axis_index("s")
        r0 = sid * ROWS_PER_TILE
        pltpu.sync_copy(zeros.at[pl.ds(r0, ROWS_PER_TILE)],
                        acc.at[pl.ds(r0, ROWS_PER_TILE)])
        plsc.subcore_barrier()

        base0 = (cid * NS + sid) * EPT

        def body(i, carry):
            base = base0 + i * CHUNK
            pltpu.sync_copy(dsti.at[pl.ds(base, CHUNK)], idx_v)
            pltpu.sync_copy(m.at[pl.ds(base, CHUNK)], rows_v)
            pltpu.sync_copy(rows_v, acc.at[idx_v], add=True)
            return carry

        lax.fori_loop(0, NCH, body, 0)
        plsc.subcore_barrier()
        pltpu.sync_copy(acc.at[pl.ds(r0, ROWS_PER_TILE)],
                        out.at[cid, pl.ds(r0, ROWS_PER_TILE)])

    return scatter


def _sc_scatter(m, dsti, zeros):
    return _sc_scatter_kernel()(m, dsti, zeros)


def _in_mlp_body(x_ref, w_ref, b_ref, wa, wb, feat_ref, tab_ref):
    f = _silu(_dot(x_ref[...], w_ref[...]) + b_ref[...])
    feat_ref[...] = f
    tab_ref[...] = jnp.concatenate([_dot(f, wa[...]), _dot(f, wb[...])], axis=1)


def _tc_in_mlp(x, w, b, wa, wb):
    return pl.pallas_call(
        _in_mlp_body,
        out_shape=(jax.ShapeDtypeStruct((N, H), F32),
                   jax.ShapeDtypeStruct((N, HW), F32)),
    )(x, w, b, wa, wb)


BE = 2048


def _edge_mlp_body(z_ref, d_ref, w1c, b1, w2, b2, wa, ba, o_ref):
    zz = _silu(z_ref[:, :H] + d_ref[...] * w1c[...] + b1[...])
    mm = _silu(_dot(zz, w2[...]) + b2[...])
    w = jax.nn.sigmoid(_dot(mm, wa[...]) + ba[...])
    mw = mm * w
    o_ref[...] = jnp.concatenate([mw, jnp.zeros_like(mw)], axis=1)


def _tc_edge_mlp(z, dpad, w1c, b1, w2, b2, wa, ba):
    grid = (E_PAD // BE,)
    full = lambda a: pl.BlockSpec(a.shape, lambda i: (0,) * a.ndim)
    return pl.pallas_call(
        _edge_mlp_body,
        grid=grid,
        in_specs=[
            pl.BlockSpec((BE, HW), lambda i: (i, 0)),
            pl.BlockSpec((BE, 1), lambda i: (i, 0)),
            full(w1c), full(b1), full(w2), full(b2), full(wa), full(ba),
        ],
        out_specs=pl.BlockSpec((BE, HW), lambda i: (i, 0)),
        out_shape=jax.ShapeDtypeStruct((E_PAD, HW), F32),
        compiler_params=pltpu.CompilerParams(dimension_semantics=("arbitrary",)),
    )(z, dpad, w1c, b1, w2, b2, wa, ba)


def _update_body(p_ref, feat_ref, w1, b1, w2, b2, wa, wb, o_ref, tab_ref):
    agg = p_ref[0, :N, :H] + p_ref[1, :N, :H]
    u = _silu(_dot(agg, w1[...]) + b1[...])
    f = _dot(u, w2[...]) + b2[...] + feat_ref[...]
    o_ref[...] = f
    tab_ref[...] = jnp.concatenate([_dot(f, wa[...]), _dot(f, wb[...])], axis=1)


def _tc_update(partial, feat, w1, b1, w2, b2, wa, wb):
    return pl.pallas_call(
        _update_body,
        out_shape=(jax.ShapeDtypeStruct((N, H), F32),
                   jax.ShapeDtypeStruct((N, HW), F32)),
    )(partial, feat, w1, b1, w2, b2, wa, wb)


def _final_body(feat_ref, nw1, nb1, nw2, nb2, ow1, ob1, ow2, ob2, o_ref):
    f = _silu(_dot(feat_ref[...], nw1[...]) + nb1[...])
    f = _dot(f, nw2[...]) + nb2[...]
    s = jnp.sum(f, axis=0, keepdims=True)
    mx = jnp.max(f, axis=0, keepdims=True)
    r = jnp.concatenate([s, s * (1.0 / N), mx], axis=1)
    h = jax.nn.relu(_dot(r, ow1[...]) + ob1[...])
    o_ref[...] = _dot(h, ow2[...]) + ob2[...]


def _tc_final(feat, nw1, nb1, nw2, nb2, ow1, ob1, ow2, ob2):
    t = ow2.shape[1]
    return pl.pallas_call(
        _final_body,
        out_shape=jax.ShapeDtypeStruct((1, t), F32),
    )(feat, nw1, nb1, nw2, nb2, ow1, ob1, ow2, ob2)


def kernel(x, edge_index, d, W_in, b_in, msg_W1, msg_b1, msg_W2, msg_b2,
           att_W, att_b, upd_W1, upd_b1, upd_W2, upd_b2, now_W1, now_b1,
           now_W2, now_b2, out_W1, out_b1, out_W2, out_b2):
    ei = edge_index.astype(jnp.int32)
    src = jnp.pad(ei[0], (0, E_PAD - E))
    dst_g = jnp.pad(ei[1], (0, E_PAD - E))
    dsti = jnp.pad(ei[1], (0, E_PAD - E), constant_values=N)
    dpad = jnp.pad(d, ((0, E_PAD - E), (0, 0)))
    zeros = jnp.zeros((NSH, HW), F32)
    wz = jnp.zeros((H, H), F32)

    w1a = [msg_W1[l, :H] for l in range(DEPTH)] + [wz]
    w1b = [msg_W1[l, H:2 * H] for l in range(DEPTH)] + [wz]

    feat, tab = _tc_in_mlp(x, W_in, b_in[None, :], w1a[0], w1b[0])

    for l in range(DEPTH):
        z = _sc_gather(tab, src, dst_g)
        m = _tc_edge_mlp(
            z, dpad,
            msg_W1[l, 2 * H:2 * H + 1], msg_b1[l][None, :],
            msg_W2[l], msg_b2[l][None, :],
            att_W[l], att_b[l][None, :],
        )
        partial = _sc_scatter(m, dsti, zeros)
        feat, tab = _tc_update(partial, feat, upd_W1[l], upd_b1[l][None, :],
                               upd_W2[l], upd_b2[l][None, :], w1a[l + 1], w1b[l + 1])

    return _tc_final(feat, now_W1, now_b1[None, :], now_W2, now_b2[None, :],
                     out_W1, out_b1[None, :], out_W2, out_b2[None, :])
